# Initial kernel scaffold; baseline (speedup 1.0000x reference)
import functools

import jax
import jax.numpy as jnp
from jax import lax
from jax.experimental import pallas as pl
from jax.experimental.pallas import tpu as pltpu
from jax.experimental.pallas import tpu_sc as plsc

_N_NODES = 10000
_N_EDGES = 320000
_C = 128
_LANES = 16
_NC = 2
_NS = 16
_NW = _NC * _NS
_FB = _C // _NW
_E_CHUNK = 2000
_N_CHUNKS = _N_EDGES // _E_CHUNK
_GROUPS = _E_CHUNK // _LANES


def _matmul_relu_body(x_ref, w_ref, b_ref, o_ref):
  acc = jnp.dot(x_ref[...], w_ref[...], preferred_element_type=jnp.float32)
  o_ref[...] = jnp.maximum(acc + b_ref[...], 0.0)


def _tc_matmul_relu(x, w, b):
  n_blk = 1000
  return pl.pallas_call(
      _matmul_relu_body,
      grid=(_N_NODES // n_blk,),
      in_specs=[
          pl.BlockSpec((n_blk, _C), lambda i: (i, 0)),
          pl.BlockSpec((_C, _C), lambda i: (0, 0)),
          pl.BlockSpec((1, _C), lambda i: (0, 0)),
      ],
      out_specs=pl.BlockSpec((n_blk, _C), lambda i: (i, 0)),
      out_shape=jax.ShapeDtypeStruct((_N_NODES, _C), jnp.float32),
  )(x, w, b.reshape(1, _C))


_mesh = plsc.VectorSubcoreMesh(core_axis_name="c", subcore_axis_name="s")


@functools.partial(
    pl.kernel,
    out_type=jax.ShapeDtypeStruct((_NW, _N_NODES, _FB), jnp.float32),
    mesh=_mesh,
    scratch_types=[
        pltpu.VMEM((_N_NODES, _FB), jnp.float32),
        pltpu.VMEM((_N_NODES, _FB), jnp.float32),
        pltpu.VMEM((_E_CHUNK,), jnp.int32),
        pltpu.VMEM((_E_CHUNK,), jnp.int32),
    ],
)
def _sc_aggregate(tbl_hbm, rows_hbm, cols_hbm, zeros_hbm, out_hbm,
                  table_v, acc_v, rows_v, cols_v):
  wid = lax.axis_index("s") * _NC + lax.axis_index("c")
  pltpu.sync_copy(tbl_hbm.at[wid], table_v)
  pltpu.sync_copy(zeros_hbm, acc_v)

  jvecs = [lax.full((_LANES,), j, jnp.int32) for j in range(_FB)]

  def chunk_body(ci, _):
    base = ci * _E_CHUNK
    pltpu.sync_copy(rows_hbm.at[pl.ds(base, _E_CHUNK)], rows_v)
    pltpu.sync_copy(cols_hbm.at[pl.ds(base, _E_CHUNK)], cols_v)

    def group_body(g, _):
      r = rows_v[pl.ds(g * _LANES, _LANES)]
      c = cols_v[pl.ds(g * _LANES, _LANES)]
      vals = [plsc.load_gather(table_v, [c, jvecs[j]]) for j in range(_FB)]
      occ, _last = plsc.scan_count(r)
      maxocc = jnp.max(occ)

      def round_body(k, _):
        m = occ == k
        for j in range(_FB):
          cur = plsc.load_gather(acc_v, [r, jvecs[j]], mask=m)
          plsc.store_scatter(acc_v, [r, jvecs[j]],
                             jnp.maximum(cur, vals[j]), mask=m)
        return 0

      lax.fori_loop(0, maxocc + 1, round_body, 0)
      return 0

    lax.fori_loop(0, _GROUPS, group_body, 0)
    return 0

  lax.fori_loop(0, _N_CHUNKS, chunk_body, 0)
  pltpu.sync_copy(acc_v, out_hbm.at[wid])


def kernel(x, edge_index, W, b):
  out = _tc_matmul_relu(x, W, b)
  tbl = out.reshape(_N_NODES, _NW, _FB).transpose(1, 0, 2)
  rows = edge_index[0]
  cols = edge_index[1]
  zeros = jnp.zeros((_N_NODES, _FB), jnp.float32)
  agg_blocked = _sc_aggregate(tbl, rows, cols, zeros)
  return agg_blocked.transpose(1, 0, 2).reshape(_N_NODES, _C)


# speedup vs baseline: 1.0011x; 1.0011x over previous
import functools

import jax
import jax.numpy as jnp
from jax import lax
from jax.experimental import pallas as pl
from jax.experimental.pallas import tpu as pltpu
from jax.experimental.pallas import tpu_sc as plsc

_N_NODES = 10000
_N_EDGES = 320000
_C = 128
_LANES = 16
_NC = 2
_NS = 16
_NW = _NC * _NS
_FB = _C // _NW
_E_CHUNK = 2000
_N_CHUNKS = _N_EDGES // _E_CHUNK
_GROUPS = _E_CHUNK // _LANES


def _matmul_relu_body(x_ref, w_ref, b_ref, o_ref):
  acc = jnp.dot(x_ref[...], w_ref[...], preferred_element_type=jnp.float32)
  o_ref[...] = jnp.maximum(acc + b_ref[...], 0.0)


def _tc_matmul_relu(x, w, b):
  n_blk = 1000
  return pl.pallas_call(
      _matmul_relu_body,
      grid=(_N_NODES // n_blk,),
      in_specs=[
          pl.BlockSpec((n_blk, _C), lambda i: (i, 0)),
          pl.BlockSpec((_C, _C), lambda i: (0, 0)),
          pl.BlockSpec((1, _C), lambda i: (0, 0)),
      ],
      out_specs=pl.BlockSpec((n_blk, _C), lambda i: (i, 0)),
      out_shape=jax.ShapeDtypeStruct((_N_NODES, _C), jnp.float32),
  )(x, w, b.reshape(1, _C))


_mesh = plsc.VectorSubcoreMesh(core_axis_name="c", subcore_axis_name="s")


@functools.partial(
    pl.kernel,
    out_type=jax.ShapeDtypeStruct((_NW, _N_NODES * _FB), jnp.float32),
    mesh=_mesh,
    compiler_params=pltpu.CompilerParams(needs_layout_passes=False),
    scratch_types=[
        pltpu.VMEM((_N_NODES * _FB,), jnp.float32),
        pltpu.VMEM((_N_NODES * _FB,), jnp.float32),
        pltpu.VMEM((_E_CHUNK,), jnp.int32),
        pltpu.VMEM((_E_CHUNK,), jnp.int32),
    ],
)
def _sc_aggregate(tbl_hbm, rows_hbm, cols_hbm, zeros_hbm, out_hbm,
                  table_v, acc_v, rows_v, cols_v):
  wid = lax.axis_index("s") * _NC + lax.axis_index("c")
  pltpu.sync_copy(tbl_hbm.at[wid], table_v)
  pltpu.sync_copy(zeros_hbm, acc_v)

  def chunk_body(ci, _):
    base = ci * _E_CHUNK
    pltpu.sync_copy(rows_hbm.at[pl.ds(base, _E_CHUNK)], rows_v)
    pltpu.sync_copy(cols_hbm.at[pl.ds(base, _E_CHUNK)], cols_v)

    def group_body(g, _):
      r = rows_v[pl.ds(g * _LANES, _LANES)]
      c = cols_v[pl.ds(g * _LANES, _LANES)]
      r4 = r * _FB
      c4 = c * _FB
      vals = [plsc.load_gather(table_v, [c4 + j]) for j in range(_FB)]
      occ, _last = plsc.scan_count(r)
      maxocc = jnp.max(occ)

      def round_body(k, _):
        m = occ == k
        for j in range(_FB):
          cur = plsc.load_gather(acc_v, [r4 + j], mask=m)
          plsc.store_scatter(acc_v, [r4 + j],
                             jnp.maximum(cur, vals[j]), mask=m)
        return 0

      lax.fori_loop(0, maxocc + 1, round_body, 0)
      return 0

    lax.fori_loop(0, _GROUPS, group_body, 0)
    return 0

  lax.fori_loop(0, _N_CHUNKS, chunk_body, 0)
  pltpu.sync_copy(acc_v, out_hbm.at[wid])


def kernel(x, edge_index, W, b):
  out = _tc_matmul_relu(x, W, b)
  tbl = out.reshape(_N_NODES, _NW, _FB).transpose(1, 0, 2)
  tbl = tbl.reshape(_NW, _N_NODES * _FB)
  rows = edge_index[0]
  cols = edge_index[1]
  zeros = jnp.zeros((_N_NODES * _FB,), jnp.float32)
  agg_blocked = _sc_aggregate(tbl, rows, cols, zeros)
  agg_blocked = agg_blocked.reshape(_NW, _N_NODES, _FB)
  return agg_blocked.transpose(1, 0, 2).reshape(_N_NODES, _C)


# speedup vs baseline: 1.9389x; 1.9367x over previous
import functools

import jax
import jax.numpy as jnp
from jax import lax
from jax.experimental import pallas as pl
from jax.experimental.pallas import tpu as pltpu
from jax.experimental.pallas import tpu_sc as plsc

_N_NODES = 10000
_N_EDGES = 320000
_C = 128
_LANES = 16
_NC = 2
_NS = 16
_NW = _NC * _NS
_FB = _C // _NW
_E_CHUNK = 2000
_N_CHUNKS = _N_EDGES // _E_CHUNK
_GROUPS = _E_CHUNK // _LANES


def _matmul_relu_t_body(x_ref, w_ref, b_ref, o_ref):
  acc = lax.dot_general(
      w_ref[...], x_ref[...],
      dimension_numbers=(((0,), (1,)), ((), ())),
      preferred_element_type=jnp.float32,
  )
  o_ref[...] = jnp.maximum(acc + b_ref[...], 0.0)


def _tc_matmul_relu_t(x, w, b):
  return pl.pallas_call(
      _matmul_relu_t_body,
      out_shape=jax.ShapeDtypeStruct((_C, _N_NODES), jnp.float32),
  )(x, w, b.reshape(_C, 1))


_mesh = plsc.VectorSubcoreMesh(core_axis_name="c", subcore_axis_name="s")


@functools.partial(
    pl.kernel,
    out_type=jax.ShapeDtypeStruct((_C, _N_NODES), jnp.float32),
    mesh=_mesh,
    compiler_params=pltpu.CompilerParams(needs_layout_passes=False),
    scratch_types=(
        [pltpu.VMEM((_N_NODES,), jnp.float32) for _ in range(_FB)]
        + [pltpu.VMEM((_N_NODES,), jnp.float32) for _ in range(_FB)]
        + [
            pltpu.VMEM((_E_CHUNK,), jnp.int32),
            pltpu.VMEM((_E_CHUNK,), jnp.int32),
        ]
    ),
)
def _sc_aggregate(tbl_hbm, rows_hbm, cols_hbm, zeros_hbm, out_hbm,
                  t0, t1, t2, t3, a0, a1, a2, a3, rows_v, cols_v):
  wid = lax.axis_index("s") * _NC + lax.axis_index("c")
  f0 = wid * _FB
  tables = (t0, t1, t2, t3)
  accs = (a0, a1, a2, a3)
  for j in range(_FB):
    pltpu.sync_copy(tbl_hbm.at[f0 + j], tables[j])
    pltpu.sync_copy(zeros_hbm, accs[j])

  def chunk_body(ci, _):
    base = ci * _E_CHUNK
    pltpu.sync_copy(rows_hbm.at[pl.ds(base, _E_CHUNK)], rows_v)
    pltpu.sync_copy(cols_hbm.at[pl.ds(base, _E_CHUNK)], cols_v)

    def group_body(g, _):
      r = rows_v[pl.ds(g * _LANES, _LANES)]
      c = cols_v[pl.ds(g * _LANES, _LANES)]
      occ, last = plsc.scan_count(r)
      vals = [plsc.load_gather(tables[j], [c]) for j in range(_FB)]
      for j in range(_FB):
        cur = plsc.load_gather(accs[j], [r])
        plsc.store_scatter(accs[j], [r], jnp.maximum(cur, vals[j]), mask=last)

      minocc = jnp.min(occ)
      maxocc = jnp.max(occ)

      @pl.when(maxocc > minocc)
      def _tail():
        def round_body(k, _):
          m = jnp.logical_and(occ == k, jnp.logical_not(last))
          for j in range(_FB):
            cur = plsc.load_gather(accs[j], [r], mask=m)
            plsc.store_scatter(accs[j], [r], jnp.maximum(cur, vals[j]),
                               mask=m)
          return 0
        lax.fori_loop(minocc, maxocc, round_body, 0)

      return 0

    lax.fori_loop(0, _GROUPS, group_body, 0)
    return 0

  lax.fori_loop(0, _N_CHUNKS, chunk_body, 0)
  for j in range(_FB):
    pltpu.sync_copy(accs[j], out_hbm.at[f0 + j])


def kernel(x, edge_index, W, b):
  out_t = _tc_matmul_relu_t(x, W, b)
  rows = edge_index[0]
  cols = edge_index[1]
  zeros = jnp.zeros((_N_NODES,), jnp.float32)
  agg_t = _sc_aggregate(out_t, rows, cols, zeros)
  return agg_t.T


# speedup vs baseline: 2.6356x; 1.3593x over previous
import functools

import jax
import jax.numpy as jnp
from jax import lax
from jax.experimental import pallas as pl
from jax.experimental.pallas import tpu as pltpu
from jax.experimental.pallas import tpu_sc as plsc

_N_NODES = 10000
_N_EDGES = 320000
_C = 128
_LANES = 16
_NC = 2
_NS = 16
_NW = _NC * _NS
_FB = _C // _NW
_E_CHUNK = 3200
_N_CHUNKS = _N_EDGES // _E_CHUNK
_UNROLL = 4
_GROUPS = _E_CHUNK // _LANES
_GROUP_ITERS = _GROUPS // _UNROLL


def _matmul_relu_t_body(x_ref, w_ref, b_ref, o_ref):
  acc = lax.dot_general(
      w_ref[...], x_ref[...],
      dimension_numbers=(((0,), (1,)), ((), ())),
      preferred_element_type=jnp.float32,
  )
  o_ref[...] = jnp.maximum(acc + b_ref[...], 0.0)


def _tc_matmul_relu_t(x, w, b):
  return pl.pallas_call(
      _matmul_relu_t_body,
      out_shape=jax.ShapeDtypeStruct((_C, _N_NODES), jnp.float32),
  )(x, w, b.reshape(_C, 1))


_mesh = plsc.VectorSubcoreMesh(core_axis_name="c", subcore_axis_name="s")


@functools.partial(
    pl.kernel,
    out_type=jax.ShapeDtypeStruct((_C, _N_NODES), jnp.float32),
    mesh=_mesh,
    compiler_params=pltpu.CompilerParams(needs_layout_passes=False),
    scratch_types=(
        [pltpu.VMEM((_N_NODES,), jnp.float32) for _ in range(_FB)]
        + [pltpu.VMEM((_N_NODES,), jnp.float32) for _ in range(_FB)]
        + [
            pltpu.VMEM((_E_CHUNK,), jnp.int32),
            pltpu.VMEM((_E_CHUNK,), jnp.int32),
            pltpu.VMEM((_E_CHUNK,), jnp.int32),
            pltpu.VMEM((_E_CHUNK,), jnp.int32),
            pltpu.SemaphoreType.DMA,
            pltpu.SemaphoreType.DMA,
            pltpu.SemaphoreType.DMA,
            pltpu.SemaphoreType.DMA,
        ]
    ),
)
def _sc_aggregate(tbl_hbm, rows_hbm, cols_hbm, zeros_hbm, out_hbm,
                  t0, t1, t2, t3, a0, a1, a2, a3,
                  rv0, cv0, rv1, cv1, sr0, sc0, sr1, sc1):
  wid = lax.axis_index("s") * _NC + lax.axis_index("c")
  f0 = wid * _FB
  tables = (t0, t1, t2, t3)
  accs = (a0, a1, a2, a3)
  for j in range(_FB):
    pltpu.sync_copy(tbl_hbm.at[f0 + j], tables[j])
    pltpu.sync_copy(zeros_hbm, accs[j])

  def start_chunk(ci, rv, cv, sr, sc_):
    base = ci * _E_CHUNK
    pltpu.async_copy(rows_hbm.at[pl.ds(base, _E_CHUNK)], rv, sr)
    pltpu.async_copy(cols_hbm.at[pl.ds(base, _E_CHUNK)], cv, sc_)

  def wait_chunk(rv, cv, sr, sc_):
    pltpu.make_async_copy(rows_hbm.at[pl.ds(0, _E_CHUNK)], rv, sr).wait()
    pltpu.make_async_copy(cols_hbm.at[pl.ds(0, _E_CHUNK)], cv, sc_).wait()

  def group_body(rv, cv, g):
    r = rv[pl.ds(g * _LANES, _LANES)]
    c = cv[pl.ds(g * _LANES, _LANES)]
    occ, last = plsc.scan_count(r)
    vals = [plsc.load_gather(tables[j], [c]) for j in range(_FB)]
    for j in range(_FB):
      cur = plsc.load_gather(accs[j], [r])
      plsc.store_scatter(accs[j], [r], jnp.maximum(cur, vals[j]), mask=last)

    minocc = jnp.min(occ)
    maxocc = jnp.max(occ)

    @pl.when(maxocc > minocc)
    def _tail():
      def round_body(k, _):
        m = jnp.logical_and(occ == k, jnp.logical_not(last))
        for j in range(_FB):
          cur = plsc.load_gather(accs[j], [r], mask=m)
          plsc.store_scatter(accs[j], [r], jnp.maximum(cur, vals[j]), mask=m)
        return 0
      lax.fori_loop(minocc, maxocc, round_body, 0)

  def compute_chunk(rv, cv):
    def group_iter(gi, _):
      for u in range(_UNROLL):
        group_body(rv, cv, gi * _UNROLL + u)
      return 0
    lax.fori_loop(0, _GROUP_ITERS, group_iter, 0)

  last_chunk = _N_CHUNKS - 1
  start_chunk(0, rv0, cv0, sr0, sc0)

  def chunk_pair(i, _):
    ci = i * 2
    start_chunk(jnp.minimum(ci + 1, last_chunk), rv1, cv1, sr1, sc1)
    wait_chunk(rv0, cv0, sr0, sc0)
    compute_chunk(rv0, cv0)
    start_chunk(jnp.minimum(ci + 2, last_chunk), rv0, cv0, sr0, sc0)
    wait_chunk(rv1, cv1, sr1, sc1)
    compute_chunk(rv1, cv1)
    return 0

  lax.fori_loop(0, _N_CHUNKS // 2, chunk_pair, 0)
  wait_chunk(rv0, cv0, sr0, sc0)

  for j in range(_FB):
    pltpu.sync_copy(accs[j], out_hbm.at[f0 + j])


def kernel(x, edge_index, W, b):
  out_t = _tc_matmul_relu_t(x, W, b)
  rows = edge_index[0]
  cols = edge_index[1]
  zeros = jnp.zeros((_N_NODES,), jnp.float32)
  agg_t = _sc_aggregate(out_t, rows, cols, zeros)
  return agg_t.T


# speedup vs baseline: 2.7183x; 1.0314x over previous
import functools

import jax
import jax.numpy as jnp
from jax import lax
from jax.experimental import pallas as pl
from jax.experimental.pallas import tpu as pltpu
from jax.experimental.pallas import tpu_sc as plsc

_N_NODES = 10000
_N_EDGES = 320000
_C = 128
_LANES = 16
_NC = 2
_NS = 16
_NW = _NC * _NS
_FB = _C // _NW
_E_CHUNK = 3200
_N_CHUNKS = _N_EDGES // _E_CHUNK
_UNROLL = 4
_GROUPS = _E_CHUNK // _LANES
_GROUP_ITERS = _GROUPS // _UNROLL


def _matmul_relu_t_body(x_ref, w_ref, b_ref, o_ref):
  acc = lax.dot_general(
      w_ref[...], x_ref[...],
      dimension_numbers=(((0,), (1,)), ((), ())),
      preferred_element_type=jnp.float32,
  )
  o_ref[...] = jnp.maximum(acc + b_ref[...], 0.0)


def _tc_matmul_relu_t(x, w, b):
  return pl.pallas_call(
      _matmul_relu_t_body,
      out_shape=jax.ShapeDtypeStruct((_C, _N_NODES), jnp.float32),
  )(x, w, b.reshape(_C, 1))


_mesh = plsc.VectorSubcoreMesh(core_axis_name="c", subcore_axis_name="s")


@functools.partial(
    pl.kernel,
    out_type=jax.ShapeDtypeStruct((_C, _N_NODES), jnp.float32),
    mesh=_mesh,
    compiler_params=pltpu.CompilerParams(needs_layout_passes=False),
    scratch_types=(
        [pltpu.VMEM((_N_NODES,), jnp.float32) for _ in range(_FB)]
        + [pltpu.VMEM((_N_NODES,), jnp.float32) for _ in range(_FB)]
        + [
            pltpu.VMEM((_E_CHUNK,), jnp.int32),
            pltpu.VMEM((_E_CHUNK,), jnp.int32),
            pltpu.VMEM((_E_CHUNK,), jnp.int32),
            pltpu.VMEM((_E_CHUNK,), jnp.int32),
            pltpu.SemaphoreType.DMA,
            pltpu.SemaphoreType.DMA,
            pltpu.SemaphoreType.DMA,
            pltpu.SemaphoreType.DMA,
        ]
    ),
)
def _sc_aggregate(tbl_hbm, rows_hbm, cols_hbm, zeros_hbm, out_hbm,
                  t0, t1, t2, t3, a0, a1, a2, a3,
                  rv0, cv0, rv1, cv1, sr0, sc0, sr1, sc1):
  wid = lax.axis_index("s") * _NC + lax.axis_index("c")
  f0 = wid * _FB
  tables = (t0, t1, t2, t3)
  accs = (a0, a1, a2, a3)
  for j in range(_FB):
    pltpu.sync_copy(tbl_hbm.at[f0 + j], tables[j])
    pltpu.sync_copy(zeros_hbm, accs[j])

  def start_chunk(ci, rv, cv, sr, sc_):
    base = ci * _E_CHUNK
    pltpu.async_copy(rows_hbm.at[pl.ds(base, _E_CHUNK)], rv, sr)
    pltpu.async_copy(cols_hbm.at[pl.ds(base, _E_CHUNK)], cv, sc_)

  def wait_chunk(rv, cv, sr, sc_):
    pltpu.make_async_copy(rows_hbm.at[pl.ds(0, _E_CHUNK)], rv, sr).wait()
    pltpu.make_async_copy(cols_hbm.at[pl.ds(0, _E_CHUNK)], cv, sc_).wait()

  def group_main(rv, cv, g):
    r = rv[pl.ds(g * _LANES, _LANES)]
    c = cv[pl.ds(g * _LANES, _LANES)]
    occ, last = plsc.scan_count(r)
    vals = [plsc.load_gather(tables[j], [c]) for j in range(_FB)]
    for j in range(_FB):
      cur = plsc.load_gather(accs[j], [r])
      plsc.store_scatter(accs[j], [r], jnp.maximum(cur, vals[j]), mask=last)
    return r, occ, last, vals

  def group_tail(r, occ, last, vals):
    maxocc = jnp.max(occ)

    @pl.when(maxocc > 1)
    def _():
      def round_body(k, _):
        m = jnp.logical_and(occ == k, jnp.logical_not(last))
        for j in range(_FB):
          cur = plsc.load_gather(accs[j], [r], mask=m)
          plsc.store_scatter(accs[j], [r], jnp.maximum(cur, vals[j]), mask=m)
        return 0
      lax.fori_loop(1, maxocc, round_body, 0)

  def compute_chunk(rv, cv):
    def group_iter(gi, _):
      states = [group_main(rv, cv, gi * _UNROLL + u) for u in range(_UNROLL)]
      occ_max = states[0][1]
      for u in range(1, _UNROLL):
        occ_max = jnp.maximum(occ_max, states[u][1])

      @pl.when(jnp.max(occ_max) > 1)
      def _():
        for u in range(_UNROLL):
          group_tail(*states[u])

      return 0
    lax.fori_loop(0, _GROUP_ITERS, group_iter, 0)

  last_chunk = _N_CHUNKS - 1
  start_chunk(0, rv0, cv0, sr0, sc0)

  def chunk_pair(i, _):
    ci = i * 2
    start_chunk(jnp.minimum(ci + 1, last_chunk), rv1, cv1, sr1, sc1)
    wait_chunk(rv0, cv0, sr0, sc0)
    compute_chunk(rv0, cv0)
    start_chunk(jnp.minimum(ci + 2, last_chunk), rv0, cv0, sr0, sc0)
    wait_chunk(rv1, cv1, sr1, sc1)
    compute_chunk(rv1, cv1)
    return 0

  lax.fori_loop(0, _N_CHUNKS // 2, chunk_pair, 0)
  wait_chunk(rv0, cv0, sr0, sc0)

  for j in range(_FB):
    pltpu.sync_copy(accs[j], out_hbm.at[f0 + j])


def kernel(x, edge_index, W, b):
  out_t = _tc_matmul_relu_t(x, W, b)
  rows = edge_index[0]
  cols = edge_index[1]
  zeros = jnp.zeros((_N_NODES,), jnp.float32)
  agg_t = _sc_aggregate(out_t, rows, cols, zeros)
  return agg_t.T


# speedup vs baseline: 3.4057x; 1.2529x over previous
import functools

import jax
import jax.numpy as jnp
from jax import lax
from jax.experimental import pallas as pl
from jax.experimental.pallas import tpu as pltpu
from jax.experimental.pallas import tpu_sc as plsc

_N_NODES = 10000
_N_EDGES = 320000
_C = 128
_LANES = 16
_NC = 2
_NS = 16
_NW = _NC * _NS
_FB = _C // _NW
_E_CHUNK = 3200
_N_CHUNKS = _N_EDGES // _E_CHUNK
_UNROLL = 4
_GROUPS = _E_CHUNK // _LANES
_GROUP_ITERS = _GROUPS // _UNROLL


def _matmul_relu_t_body(x_ref, w_ref, b_ref, o_ref):
  acc = lax.dot_general(
      w_ref[...], x_ref[...],
      dimension_numbers=(((0,), (1,)), ((), ())),
      preferred_element_type=jnp.float32,
  )
  o_ref[...] = jnp.maximum(acc + b_ref[...], 0.0)


def _tc_matmul_relu_t(x, w, b):
  return pl.pallas_call(
      _matmul_relu_t_body,
      out_shape=jax.ShapeDtypeStruct((_C, _N_NODES), jnp.float32),
  )(x, w, b.reshape(_C, 1))


_mesh = plsc.VectorSubcoreMesh(core_axis_name="c", subcore_axis_name="s")


@functools.partial(
    pl.kernel,
    out_type=jax.ShapeDtypeStruct((_C, _N_NODES), jnp.float32),
    mesh=_mesh,
    compiler_params=pltpu.CompilerParams(needs_layout_passes=False),
    scratch_types=(
        [pltpu.VMEM((_N_NODES,), jnp.float32) for _ in range(_FB)]
        + [pltpu.VMEM((_N_NODES,), jnp.float32) for _ in range(_FB)]
        + [
            pltpu.VMEM((_E_CHUNK,), jnp.int32),
            pltpu.VMEM((_E_CHUNK,), jnp.int32),
            pltpu.VMEM((_E_CHUNK,), jnp.int32),
            pltpu.VMEM((_E_CHUNK,), jnp.int32),
            pltpu.SemaphoreType.DMA,
            pltpu.SemaphoreType.DMA,
            pltpu.SemaphoreType.DMA,
            pltpu.SemaphoreType.DMA,
        ]
    ),
)
def _sc_aggregate(tbl_hbm, rows_hbm, cols_hbm, zeros_hbm, out_hbm,
                  t0, t1, t2, t3, a0, a1, a2, a3,
                  rv0, cv0, rv1, cv1, sr0, sc0, sr1, sc1):
  wid = lax.axis_index("s") * _NC + lax.axis_index("c")
  f0 = wid * _FB
  tables = (t0, t1, t2, t3)
  accs = (a0, a1, a2, a3)
  for j in range(_FB):
    pltpu.sync_copy(tbl_hbm.at[f0 + j], tables[j])
    pltpu.sync_copy(zeros_hbm, accs[j])

  def start_chunk(ci, rv, cv, sr, sc_):
    base = ci * _E_CHUNK
    pltpu.async_copy(rows_hbm.at[pl.ds(base, _E_CHUNK)], rv, sr)
    pltpu.async_copy(cols_hbm.at[pl.ds(base, _E_CHUNK)], cv, sc_)

  def wait_chunk(rv, cv, sr, sc_):
    pltpu.make_async_copy(rows_hbm.at[pl.ds(0, _E_CHUNK)], rv, sr).wait()
    pltpu.make_async_copy(cols_hbm.at[pl.ds(0, _E_CHUNK)], cv, sc_).wait()

  def group_main(rv, cv, g):
    r = rv[pl.ds(g * _LANES, _LANES)]
    c = cv[pl.ds(g * _LANES, _LANES)]
    occ, last = plsc.scan_count(r)
    vals = [plsc.load_gather(tables[j], [c]) for j in range(_FB)]
    curs = [plsc.load_gather(accs[j], [r]) for j in range(_FB)]
    for j in range(_FB):
      plsc.store_scatter(accs[j], [r], jnp.maximum(curs[j], vals[j]),
                         mask=last)
    return r, occ, last, vals

  def group_tail(r, occ, last, vals):
    maxocc = jnp.max(occ)

    @pl.when(maxocc > 1)
    def _():
      def round_body(k, _):
        m = jnp.logical_and(occ == k, jnp.logical_not(last))
        for j in range(_FB):
          cur = plsc.load_gather(accs[j], [r], mask=m)
          plsc.store_scatter(accs[j], [r], jnp.maximum(cur, vals[j]), mask=m)
        return 0
      lax.fori_loop(1, maxocc, round_body, 0)

  def compute_chunk(rv, cv):
    def group_iter(gi, _):
      states = [group_main(rv, cv, gi * _UNROLL + u) for u in range(_UNROLL)]
      occ_max = states[0][1]
      for u in range(1, _UNROLL):
        occ_max = jnp.maximum(occ_max, states[u][1])

      @pl.when(jnp.max(occ_max) > 1)
      def _():
        for u in range(_UNROLL):
          group_tail(*states[u])

      return 0
    lax.fori_loop(0, _GROUP_ITERS, group_iter, 0)

  last_chunk = _N_CHUNKS - 1
  start_chunk(0, rv0, cv0, sr0, sc0)

  def chunk_pair(i, _):
    ci = i * 2
    start_chunk(jnp.minimum(ci + 1, last_chunk), rv1, cv1, sr1, sc1)
    wait_chunk(rv0, cv0, sr0, sc0)
    compute_chunk(rv0, cv0)
    start_chunk(jnp.minimum(ci + 2, last_chunk), rv0, cv0, sr0, sc0)
    wait_chunk(rv1, cv1, sr1, sc1)
    compute_chunk(rv1, cv1)
    return 0

  lax.fori_loop(0, _N_CHUNKS // 2, chunk_pair, 0)
  wait_chunk(rv0, cv0, sr0, sc0)

  for j in range(_FB):
    pltpu.sync_copy(accs[j], out_hbm.at[f0 + j])


def kernel(x, edge_index, W, b):
  out_t = _tc_matmul_relu_t(x, W, b)
  rows = edge_index[0]
  cols = edge_index[1]
  zeros = jnp.zeros((_N_NODES,), jnp.float32)
  agg_t = _sc_aggregate(out_t, rows, cols, zeros)
  return agg_t.T
